# baseline (speedup 1.0000x reference)
import functools

import jax
import jax.numpy as jnp
from jax import lax
from jax.experimental import pallas as pl
from jax.experimental.pallas import tpu as pltpu
from jax.experimental.pallas import tpu_sc as plsc

_N = 10000
_E = 320000
_D = 128
_T = 4
_STEPS = 8

_NC = 2
_NS = 16
_NW = _NC * _NS
_CHUNK = 128
_NCH = 80
_EPW = _NCH * _CHUNK
_EPAD = _NW * _EPW
_ASH_ROWS = 10240
_TRASH = _N

_NB = 2000
_GRID = _N // _NB


def _init_body(x_ref, wlin_ref, blin_ref, wet_ref, bet_ref, h_ref, p_ref):
    h = jnp.dot(x_ref[...], wlin_ref[...], preferred_element_type=jnp.float32)
    h = h + blin_ref[...]
    h_ref[...] = h
    for e in range(_T):
        p_ref[e] = (
            jnp.dot(h, wet_ref[e], preferred_element_type=jnp.float32)
            + bet_ref[e]
        )


def _tc_init(x, wlin_t, blin, wet_t, bet):
    return pl.pallas_call(
        _init_body,
        grid=(_GRID,),
        in_specs=[
            pl.BlockSpec((_NB, _D), lambda i: (i, 0)),
            pl.BlockSpec((_D, _D), lambda i: (0, 0)),
            pl.BlockSpec((1, _D), lambda i: (0, 0)),
            pl.BlockSpec((_T, _D, _D), lambda i: (0, 0, 0)),
            pl.BlockSpec((_T, 1, _D), lambda i: (0, 0, 0)),
        ],
        out_specs=[
            pl.BlockSpec((_NB, _D), lambda i: (i, 0)),
            pl.BlockSpec((_T, _NB, _D), lambda i: (0, i, 0)),
        ],
        out_shape=[
            jax.ShapeDtypeStruct((_N, _D), jnp.float32),
            jax.ShapeDtypeStruct((_T, _N, _D), jnp.float32),
        ],
    )(x, wlin_t, blin, wet_t, bet)


def _gru_body(ap_ref, h_ref, wih_ref, bih_ref, whh_ref, bhh_ref,
              wet_ref, bet_ref, hout_ref, pout_ref):
    a = ap_ref[0] + ap_ref[1]
    h = h_ref[...]
    gi = jnp.dot(a, wih_ref[...], preferred_element_type=jnp.float32)
    gi = gi + bih_ref[...]
    gh = jnp.dot(h, whh_ref[...], preferred_element_type=jnp.float32)
    gh = gh + bhh_ref[...]
    r = jax.nn.sigmoid(gi[:, 0:_D] + gh[:, 0:_D])
    z = jax.nn.sigmoid(gi[:, _D:2 * _D] + gh[:, _D:2 * _D])
    n = jnp.tanh(gi[:, 2 * _D:3 * _D] + r * gh[:, 2 * _D:3 * _D])
    hn = (1.0 - z) * n + z * h
    hout_ref[...] = hn
    for e in range(_T):
        pout_ref[e] = (
            jnp.dot(hn, wet_ref[e], preferred_element_type=jnp.float32)
            + bet_ref[e]
        )


def _tc_gru(apart, h, wih_t, bih, whh_t, bhh, wet_t, bet):
    return pl.pallas_call(
        _gru_body,
        grid=(_GRID,),
        in_specs=[
            pl.BlockSpec((_NC, _NB, _D), lambda i: (0, i, 0)),
            pl.BlockSpec((_NB, _D), lambda i: (i, 0)),
            pl.BlockSpec((_D, 3 * _D), lambda i: (0, 0)),
            pl.BlockSpec((1, 3 * _D), lambda i: (0, 0)),
            pl.BlockSpec((_D, 3 * _D), lambda i: (0, 0)),
            pl.BlockSpec((1, 3 * _D), lambda i: (0, 0)),
            pl.BlockSpec((_T, _D, _D), lambda i: (0, 0, 0)),
            pl.BlockSpec((_T, 1, _D), lambda i: (0, 0, 0)),
        ],
        out_specs=[
            pl.BlockSpec((_NB, _D), lambda i: (i, 0)),
            pl.BlockSpec((_T, _NB, _D), lambda i: (0, i, 0)),
        ],
        out_shape=[
            jax.ShapeDtypeStruct((_N, _D), jnp.float32),
            jax.ShapeDtypeStruct((_T, _N, _D), jnp.float32),
        ],
    )(apart, h, wih_t, bih, whh_t, bhh, wet_t, bet)


def _readout_body(h_ref, wcls_ref, bcls_ref, out_ref):
    i = pl.program_id(0)
    s = jnp.sum(h_ref[...], axis=0, keepdims=True)
    part = jnp.dot(s, wcls_ref[...], preferred_element_type=jnp.float32)

    @pl.when(i == 0)
    def _():
        out_ref[...] = bcls_ref[...]

    out_ref[...] += part * (1.0 / _N)


def _tc_readout(h, wcls_t, bcls):
    return pl.pallas_call(
        _readout_body,
        grid=(_GRID,),
        in_specs=[
            pl.BlockSpec((_NB, _D), lambda i: (i, 0)),
            pl.BlockSpec((_D, 2), lambda i: (0, 0)),
            pl.BlockSpec((1, 2), lambda i: (0, 0)),
        ],
        out_specs=pl.BlockSpec((1, 2), lambda i: (0, 0)),
        out_shape=jax.ShapeDtypeStruct((1, 2), jnp.float32),
    )(h, wcls_t, bcls)


@functools.partial(
    pl.kernel,
    out_type=jax.ShapeDtypeStruct((_NC, _ASH_ROWS, _D), jnp.float32),
    mesh=plsc.VectorSubcoreMesh(core_axis_name="c", subcore_axis_name="s"),
    scratch_types=[
        pltpu.VMEM((_CHUNK,), jnp.int32),
        pltpu.VMEM((_CHUNK,), jnp.int32),
        pltpu.VMEM((_CHUNK,), jnp.int32),
        pltpu.VMEM((_CHUNK,), jnp.int32),
        pltpu.VMEM((_CHUNK, _D), jnp.float32),
        pltpu.VMEM((_CHUNK, _D), jnp.float32),
        pltpu.VMEM((16, _D), jnp.float32),
        pltpu.VMEM_SHARED((_ASH_ROWS, _D), jnp.float32),
        pltpu.SemaphoreType.DMA,
        pltpu.SemaphoreType.DMA,
    ],
)
def _sc_scatter(p_hbm, gidx_hbm, dst_hbm, out_hbm,
                gi_a, gi_b, ds_a, ds_b, rows_a, rows_b, zbuf, a_sh,
                gsem, lsem):
    c = lax.axis_index("c")
    s = lax.axis_index("s")
    wid = c * _NS + s

    for r in range(16):
        for q in range(8):
            zbuf[r, pl.ds(q * 16, 16)] = jnp.zeros((16,), jnp.float32)
    zrows = _ASH_ROWS // _NS
    zc = [
        pltpu.async_copy(zbuf, a_sh.at[pl.ds(s * zrows + k * 16, 16)], lsem)
        for k in range(zrows // 16)
    ]
    for copy in zc:
        copy.wait()
    plsc.subcore_barrier()

    base = wid * _EPW

    def load_idx(j, gi, ds):
        pltpu.sync_copy(gidx_hbm.at[pl.ds(base + j * _CHUNK, _CHUNK)], gi)
        pltpu.sync_copy(dst_hbm.at[pl.ds(base + j * _CHUNK, _CHUNK)], ds)

    load_idx(0, gi_a, ds_a)
    pltpu.async_copy(p_hbm.at[gi_a], rows_a, gsem)

    def pair_step(g, carry):
        load_idx(2 * g + 1, gi_b, ds_b)
        pltpu.make_async_copy(p_hbm.at[gi_a], rows_a, gsem).wait()
        pltpu.async_copy(p_hbm.at[gi_b], rows_b, gsem)
        pltpu.sync_copy(rows_a, a_sh.at[ds_a], add=True)

        @pl.when(g + 1 < _NCH // 2)
        def _():
            load_idx(2 * g + 2, gi_a, ds_a)

        pltpu.make_async_copy(p_hbm.at[gi_b], rows_b, gsem).wait()

        @pl.when(g + 1 < _NCH // 2)
        def _():
            pltpu.async_copy(p_hbm.at[gi_a], rows_a, gsem)

        pltpu.sync_copy(rows_b, a_sh.at[ds_b], add=True)
        return carry

    lax.fori_loop(0, _NCH // 2, pair_step, 0)
    plsc.subcore_barrier()

    rows_per = _ASH_ROWS // _NS
    pltpu.sync_copy(
        a_sh.at[pl.ds(s * rows_per, rows_per)],
        out_hbm.at[c, pl.ds(s * rows_per, rows_per)],
    )


def kernel(x, edge_index, edge_attr, W_lin, b_lin, W_et, b_et,
           W_ih, b_ih, W_hh, b_hh, W_cls, b_cls):
    src = edge_index[0]
    dst = edge_index[1]
    gidx = edge_attr * _N + src

    pad = _EPAD - _E
    gidx_p = jnp.concatenate([gidx, jnp.zeros((pad,), jnp.int32)])
    dst_p = jnp.concatenate([dst, jnp.full((pad,), _TRASH, jnp.int32)])

    wlin_t = W_lin.T
    blin = b_lin.reshape(1, _D)
    wet_t = jnp.transpose(W_et, (0, 2, 1))
    bet = b_et.reshape(_T, 1, _D)
    wih_t = W_ih.T
    bih = b_ih.reshape(1, 3 * _D)
    whh_t = W_hh.T
    bhh = b_hh.reshape(1, 3 * _D)
    wcls_t = W_cls.T
    bcls = b_cls.reshape(1, 2)

    h, p = _tc_init(x, wlin_t, blin, wet_t, bet)
    for _ in range(_STEPS):
        apart = _sc_scatter(p.reshape(_T * _N, _D), gidx_p, dst_p)
        h, p = _tc_gru(apart, h, wih_t, bih, whh_t, bhh, wet_t, bet)
    return _tc_readout(h, wcls_t, bcls)


# speedup vs baseline: 1.0065x; 1.0065x over previous
import functools

import jax
import jax.numpy as jnp
from jax import lax
from jax.experimental import pallas as pl
from jax.experimental.pallas import tpu as pltpu
from jax.experimental.pallas import tpu_sc as plsc

_N = 10000
_E = 320000
_D = 128
_T = 4
_STEPS = 8

_NC = 2
_NS = 16
_NW = _NC * _NS
_CHUNK = 128
_NCH = 80
_EPW = _NCH * _CHUNK
_EPAD = _NW * _EPW
_ASH_ROWS = 10240
_TRASH = _N

_NB = 2000
_GRID = _N // _NB


def _init_body(x_ref, wlin_ref, blin_ref, wet_ref, bet_ref, h_ref, p_ref):
    h = jnp.dot(x_ref[...], wlin_ref[...], preferred_element_type=jnp.float32)
    h = h + blin_ref[...]
    h_ref[...] = h
    for e in range(_T):
        p_ref[e] = (
            jnp.dot(h, wet_ref[e], preferred_element_type=jnp.float32)
            + bet_ref[e]
        )


def _tc_init(x, wlin_t, blin, wet_t, bet):
    return pl.pallas_call(
        _init_body,
        grid=(_GRID,),
        in_specs=[
            pl.BlockSpec((_NB, _D), lambda i: (i, 0)),
            pl.BlockSpec((_D, _D), lambda i: (0, 0)),
            pl.BlockSpec((1, _D), lambda i: (0, 0)),
            pl.BlockSpec((_T, _D, _D), lambda i: (0, 0, 0)),
            pl.BlockSpec((_T, 1, _D), lambda i: (0, 0, 0)),
        ],
        out_specs=[
            pl.BlockSpec((_NB, _D), lambda i: (i, 0)),
            pl.BlockSpec((_T, _NB, _D), lambda i: (0, i, 0)),
        ],
        out_shape=[
            jax.ShapeDtypeStruct((_N, _D), jnp.float32),
            jax.ShapeDtypeStruct((_T, _N, _D), jnp.float32),
        ],
    )(x, wlin_t, blin, wet_t, bet)


def _gru_body(ap_ref, h_ref, wih_ref, bih_ref, whh_ref, bhh_ref,
              wet_ref, bet_ref, hout_ref, pout_ref):
    a = ap_ref[0] + ap_ref[1]
    h = h_ref[...]
    gi = jnp.dot(a, wih_ref[...], preferred_element_type=jnp.float32)
    gi = gi + bih_ref[...]
    gh = jnp.dot(h, whh_ref[...], preferred_element_type=jnp.float32)
    gh = gh + bhh_ref[...]
    r = jax.nn.sigmoid(gi[:, 0:_D] + gh[:, 0:_D])
    z = jax.nn.sigmoid(gi[:, _D:2 * _D] + gh[:, _D:2 * _D])
    n = jnp.tanh(gi[:, 2 * _D:3 * _D] + r * gh[:, 2 * _D:3 * _D])
    hn = (1.0 - z) * n + z * h
    hout_ref[...] = hn
    for e in range(_T):
        pout_ref[e] = (
            jnp.dot(hn, wet_ref[e], preferred_element_type=jnp.float32)
            + bet_ref[e]
        )


def _tc_gru(apart, h, wih_t, bih, whh_t, bhh, wet_t, bet):
    return pl.pallas_call(
        _gru_body,
        grid=(_GRID,),
        in_specs=[
            pl.BlockSpec((_NC, _NB, _D), lambda i: (0, i, 0)),
            pl.BlockSpec((_NB, _D), lambda i: (i, 0)),
            pl.BlockSpec((_D, 3 * _D), lambda i: (0, 0)),
            pl.BlockSpec((1, 3 * _D), lambda i: (0, 0)),
            pl.BlockSpec((_D, 3 * _D), lambda i: (0, 0)),
            pl.BlockSpec((1, 3 * _D), lambda i: (0, 0)),
            pl.BlockSpec((_T, _D, _D), lambda i: (0, 0, 0)),
            pl.BlockSpec((_T, 1, _D), lambda i: (0, 0, 0)),
        ],
        out_specs=[
            pl.BlockSpec((_NB, _D), lambda i: (i, 0)),
            pl.BlockSpec((_T, _NB, _D), lambda i: (0, i, 0)),
        ],
        out_shape=[
            jax.ShapeDtypeStruct((_N, _D), jnp.float32),
            jax.ShapeDtypeStruct((_T, _N, _D), jnp.float32),
        ],
    )(apart, h, wih_t, bih, whh_t, bhh, wet_t, bet)


def _readout_body(h_ref, wcls_ref, bcls_ref, out_ref):
    i = pl.program_id(0)
    s = jnp.sum(h_ref[...], axis=0, keepdims=True)
    part = jnp.dot(s, wcls_ref[...], preferred_element_type=jnp.float32)

    @pl.when(i == 0)
    def _():
        out_ref[...] = bcls_ref[...]

    out_ref[...] += part * (1.0 / _N)


def _tc_readout(h, wcls_t, bcls):
    return pl.pallas_call(
        _readout_body,
        grid=(_GRID,),
        in_specs=[
            pl.BlockSpec((_NB, _D), lambda i: (i, 0)),
            pl.BlockSpec((_D, 2), lambda i: (0, 0)),
            pl.BlockSpec((1, 2), lambda i: (0, 0)),
        ],
        out_specs=pl.BlockSpec((1, 2), lambda i: (0, 0)),
        out_shape=jax.ShapeDtypeStruct((1, 2), jnp.float32),
    )(h, wcls_t, bcls)


@functools.partial(
    pl.kernel,
    out_type=jax.ShapeDtypeStruct((_NC, _ASH_ROWS, _D), jnp.float32),
    mesh=plsc.VectorSubcoreMesh(core_axis_name="c", subcore_axis_name="s"),
    scratch_types=(
        [pltpu.VMEM((_CHUNK,), jnp.int32)] * 4 +
        [pltpu.VMEM((_CHUNK,), jnp.int32)] * 4 +
        [pltpu.VMEM((_CHUNK, _D), jnp.float32)] * 2 +
        [
            pltpu.VMEM((16, _D), jnp.float32),
            pltpu.VMEM_SHARED((_ASH_ROWS, _D), jnp.float32),
            pltpu.SemaphoreType.DMA,
        ] +
        [pltpu.SemaphoreType.DMA] * 4 +
        [pltpu.SemaphoreType.DMA] * 4 +
        [pltpu.SemaphoreType.DMA]
    ),
)
def _sc_scatter(p_hbm, gidx_hbm, dst_hbm, out_hbm,
                gi0, gi1, gi2, gi3, ds0, ds1, ds2, ds3, rows0, rows1,
                zbuf, a_sh, gsem,
                gis0, gis1, gis2, gis3, dss0, dss1, dss2, dss3, lsem):
    gi = [gi0, gi1, gi2, gi3]
    ds = [ds0, ds1, ds2, ds3]
    rows = [rows0, rows1]
    gisem = [gis0, gis1, gis2, gis3]
    dsem = [dss0, dss1, dss2, dss3]
    c = lax.axis_index("c")
    s = lax.axis_index("s")
    wid = c * _NS + s
    base = wid * _EPW

    for r in range(16):
        for q in range(8):
            zbuf[r, pl.ds(q * 16, 16)] = jnp.zeros((16,), jnp.float32)
    zrows = _ASH_ROWS // _NS
    zc = [
        pltpu.async_copy(zbuf, a_sh.at[pl.ds(s * zrows + k * 16, 16)], lsem)
        for k in range(zrows // 16)
    ]
    for copy in zc:
        copy.wait()
    plsc.subcore_barrier()


    def gi_load(j, k):
        return pltpu.async_copy(
            gidx_hbm.at[pl.ds(base + j * _CHUNK, _CHUNK)], gi[k], gisem[k])

    def ds_load(j, k):
        return pltpu.async_copy(
            dst_hbm.at[pl.ds(base + j * _CHUNK, _CHUNK)], ds[k], dsem[k])

    def gi_drain(j, k):
        pltpu.make_async_copy(
            gidx_hbm.at[pl.ds(base + j * _CHUNK, _CHUNK)], gi[k],
            gisem[k]).wait()

    def ds_drain(j, k):
        pltpu.make_async_copy(
            dst_hbm.at[pl.ds(base + j * _CHUNK, _CHUNK)], ds[k],
            dsem[k]).wait()

    def gather_start(j, k, b):
        pltpu.async_copy(p_hbm.at[gi[k]], rows[b], gsem)

    def gather_wait(j, k, b):
        pltpu.make_async_copy(p_hbm.at[gi[k]], rows[b], gsem).wait()

    for k in range(4):
        gi_load(k, k)
        ds_load(k, k)
    gi_drain(0, 0)
    gather_start(0, 0, 0)

    def quad_step(g, carry):
        for k in range(4):
            j = 4 * g + k
            gather_wait(j, k, k % 2)

            @pl.when(j + 1 < _NCH)
            def _(k=k):
                gi_drain(j + 1, (k + 1) % 4)
                gather_start(j + 1, (k + 1) % 4, (k + 1) % 2)

            ds_drain(j, k)
            pltpu.sync_copy(rows[k % 2], a_sh.at[ds[k]], add=True)

            @pl.when(j + 4 < _NCH)
            def _(k=k):
                gi_load(j + 4, k)
                ds_load(j + 4, k)
        return carry

    lax.fori_loop(0, _NCH // 4, quad_step, 0)
    plsc.subcore_barrier()

    rows_per = _ASH_ROWS // _NS
    pltpu.sync_copy(
        a_sh.at[pl.ds(s * rows_per, rows_per)],
        out_hbm.at[c, pl.ds(s * rows_per, rows_per)],
    )


def kernel(x, edge_index, edge_attr, W_lin, b_lin, W_et, b_et,
           W_ih, b_ih, W_hh, b_hh, W_cls, b_cls):
    src = edge_index[0]
    dst = edge_index[1]
    gidx = edge_attr * _N + src

    pad = _EPAD - _E
    gidx_p = jnp.concatenate([gidx, jnp.zeros((pad,), jnp.int32)])
    dst_p = jnp.concatenate([dst, jnp.full((pad,), _TRASH, jnp.int32)])

    wlin_t = W_lin.T
    blin = b_lin.reshape(1, _D)
    wet_t = jnp.transpose(W_et, (0, 2, 1))
    bet = b_et.reshape(_T, 1, _D)
    wih_t = W_ih.T
    bih = b_ih.reshape(1, 3 * _D)
    whh_t = W_hh.T
    bhh = b_hh.reshape(1, 3 * _D)
    wcls_t = W_cls.T
    bcls = b_cls.reshape(1, 2)

    h, p = _tc_init(x, wlin_t, blin, wet_t, bet)
    for _ in range(_STEPS):
        apart = _sc_scatter(p.reshape(_T * _N, _D), gidx_p, dst_p)
        h, p = _tc_gru(apart, h, wih_t, bih, whh_t, bhh, wet_t, bet)
    return _tc_readout(h, wcls_t, bcls)


# speedup vs baseline: 1.0765x; 1.0695x over previous
import functools

import jax
import jax.numpy as jnp
from jax import lax
from jax.experimental import pallas as pl
from jax.experimental.pallas import tpu as pltpu
from jax.experimental.pallas import tpu_sc as plsc

_N = 10000
_E = 320000
_D = 128
_T = 4
_STEPS = 8

_NC = 2
_NS = 16
_NW = _NC * _NS
_CHUNK = 128
_NCH = 80
_EPW = _NCH * _CHUNK
_EPAD = _NW * _EPW
_ASH_ROWS = 10240
_TRASH = _N

_NB = 2000
_GRID = _N // _NB


def _init_body(x_ref, wlin_ref, blin_ref, wet_ref, bet_ref, h_ref, p_ref):
    h = jnp.dot(x_ref[...], wlin_ref[...], preferred_element_type=jnp.float32)
    h = h + blin_ref[...]
    h_ref[...] = h
    for e in range(_T):
        p_ref[e] = (
            jnp.dot(h, wet_ref[e], preferred_element_type=jnp.float32)
            + bet_ref[e]
        )


def _tc_init(x, wlin_t, blin, wet_t, bet):
    return pl.pallas_call(
        _init_body,
        grid=(_GRID,),
        in_specs=[
            pl.BlockSpec((_NB, _D), lambda i: (i, 0)),
            pl.BlockSpec((_D, _D), lambda i: (0, 0)),
            pl.BlockSpec((1, _D), lambda i: (0, 0)),
            pl.BlockSpec((_T, _D, _D), lambda i: (0, 0, 0)),
            pl.BlockSpec((_T, 1, _D), lambda i: (0, 0, 0)),
        ],
        out_specs=[
            pl.BlockSpec((_NB, _D), lambda i: (i, 0)),
            pl.BlockSpec((_T, _NB, _D), lambda i: (0, i, 0)),
        ],
        out_shape=[
            jax.ShapeDtypeStruct((_N, _D), jnp.float32),
            jax.ShapeDtypeStruct((_T, _N, _D), jnp.float32),
        ],
    )(x, wlin_t, blin, wet_t, bet)


def _gru_body(ap_ref, h_ref, wih_ref, bih_ref, whh_ref, bhh_ref,
              wet_ref, bet_ref, hout_ref, pout_ref):
    a = ap_ref[0] + ap_ref[1]
    h = h_ref[...]
    gi = jnp.dot(a, wih_ref[...], preferred_element_type=jnp.float32)
    gi = gi + bih_ref[...]
    gh = jnp.dot(h, whh_ref[...], preferred_element_type=jnp.float32)
    gh = gh + bhh_ref[...]
    r = jax.nn.sigmoid(gi[:, 0:_D] + gh[:, 0:_D])
    z = jax.nn.sigmoid(gi[:, _D:2 * _D] + gh[:, _D:2 * _D])
    n = jnp.tanh(gi[:, 2 * _D:3 * _D] + r * gh[:, 2 * _D:3 * _D])
    hn = (1.0 - z) * n + z * h
    hout_ref[...] = hn
    for e in range(_T):
        pout_ref[e] = (
            jnp.dot(hn, wet_ref[e], preferred_element_type=jnp.float32)
            + bet_ref[e]
        )


def _tc_gru(apart, h, wih_t, bih, whh_t, bhh, wet_t, bet):
    return pl.pallas_call(
        _gru_body,
        grid=(_GRID,),
        in_specs=[
            pl.BlockSpec((_NC, _NB, _D), lambda i: (0, i, 0)),
            pl.BlockSpec((_NB, _D), lambda i: (i, 0)),
            pl.BlockSpec((_D, 3 * _D), lambda i: (0, 0)),
            pl.BlockSpec((1, 3 * _D), lambda i: (0, 0)),
            pl.BlockSpec((_D, 3 * _D), lambda i: (0, 0)),
            pl.BlockSpec((1, 3 * _D), lambda i: (0, 0)),
            pl.BlockSpec((_T, _D, _D), lambda i: (0, 0, 0)),
            pl.BlockSpec((_T, 1, _D), lambda i: (0, 0, 0)),
        ],
        out_specs=[
            pl.BlockSpec((_NB, _D), lambda i: (i, 0)),
            pl.BlockSpec((_T, _NB, _D), lambda i: (0, i, 0)),
        ],
        out_shape=[
            jax.ShapeDtypeStruct((_N, _D), jnp.float32),
            jax.ShapeDtypeStruct((_T, _N, _D), jnp.float32),
        ],
    )(apart, h, wih_t, bih, whh_t, bhh, wet_t, bet)


def _readout_body(h_ref, wcls_ref, bcls_ref, out_ref):
    i = pl.program_id(0)
    s = jnp.sum(h_ref[...], axis=0, keepdims=True)
    part = jnp.dot(s, wcls_ref[...], preferred_element_type=jnp.float32)

    @pl.when(i == 0)
    def _():
        out_ref[...] = bcls_ref[...]

    out_ref[...] += part * (1.0 / _N)


def _tc_readout(h, wcls_t, bcls):
    return pl.pallas_call(
        _readout_body,
        grid=(_GRID,),
        in_specs=[
            pl.BlockSpec((_NB, _D), lambda i: (i, 0)),
            pl.BlockSpec((_D, 2), lambda i: (0, 0)),
            pl.BlockSpec((1, 2), lambda i: (0, 0)),
        ],
        out_specs=pl.BlockSpec((1, 2), lambda i: (0, 0)),
        out_shape=jax.ShapeDtypeStruct((1, 2), jnp.float32),
    )(h, wcls_t, bcls)


@functools.partial(
    pl.kernel,
    out_type=jax.ShapeDtypeStruct((_NC, _ASH_ROWS, _D), jnp.float32),
    mesh=plsc.VectorSubcoreMesh(core_axis_name="c", subcore_axis_name="s"),
    scratch_types=[
        pltpu.VMEM((16, _CHUNK), jnp.int32),
        pltpu.VMEM((16, _CHUNK), jnp.int32),
        pltpu.VMEM((_CHUNK, _D), jnp.float32),
        pltpu.VMEM((_CHUNK, _D), jnp.float32),
        pltpu.VMEM((16, _D), jnp.float32),
        pltpu.VMEM_SHARED((_ASH_ROWS, _D), jnp.float32),
        pltpu.SemaphoreType.DMA,
        pltpu.SemaphoreType.DMA,
        pltpu.SemaphoreType.DMA,
        pltpu.SemaphoreType.DMA,
    ],
)
def _sc_scatter(p_hbm, idx_hbm, out_hbm,
                bank_a, bank_b, rows0, rows1, zbuf, a_sh,
                gsem, sem_a, sem_b, lsem):
    rows = [rows0, rows1]
    c = lax.axis_index("c")
    s = lax.axis_index("s")
    wid = c * _NS + s
    ngrp = _NCH // 8
    gbase = wid * ngrp

    pltpu.async_copy(idx_hbm.at[gbase], bank_a, sem_a)
    pltpu.async_copy(idx_hbm.at[gbase + 1], bank_b, sem_b)
    for r in range(16):
        for q in range(8):
            zbuf[r, pl.ds(q * 16, 16)] = jnp.zeros((16,), jnp.float32)
    zrows = _ASH_ROWS // _NS
    zc = [
        pltpu.async_copy(zbuf, a_sh.at[pl.ds(s * zrows + k * 16, 16)], lsem)
        for k in range(zrows // 16)
    ]
    for copy in zc:
        copy.wait()
    plsc.subcore_barrier()


    def bank_drain(grp, bank, sem):
        pltpu.make_async_copy(idx_hbm.at[gbase + grp], bank, sem).wait()

    def gather_start(bank, q, b):
        pltpu.async_copy(p_hbm.at[bank.at[q]], rows[b], gsem)

    def gather_wait(bank, q, b):
        pltpu.make_async_copy(p_hbm.at[bank.at[q]], rows[b], gsem).wait()

    bank_drain(0, bank_a, sem_a)
    gather_start(bank_a, 0, 0)

    def dgroup_step(G, carry):
        for q in range(7):
            gather_wait(bank_a, q, q % 2)
            gather_start(bank_a, q + 1, (q + 1) % 2)
            pltpu.sync_copy(rows[q % 2], a_sh.at[bank_a.at[8 + q]], add=True)
        gather_wait(bank_a, 7, 1)
        bank_drain(2 * G + 1, bank_b, sem_b)
        gather_start(bank_b, 0, 0)
        pltpu.sync_copy(rows[1], a_sh.at[bank_a.at[15]], add=True)

        @pl.when(2 * G + 2 < ngrp)
        def _():
            pltpu.async_copy(idx_hbm.at[gbase + 2 * G + 2], bank_a, sem_a)

        for q in range(7):
            gather_wait(bank_b, q, q % 2)
            gather_start(bank_b, q + 1, (q + 1) % 2)
            pltpu.sync_copy(rows[q % 2], a_sh.at[bank_b.at[8 + q]], add=True)
        gather_wait(bank_b, 7, 1)

        @pl.when(2 * G + 2 < ngrp)
        def _():
            bank_drain(2 * G + 2, bank_a, sem_a)
            gather_start(bank_a, 0, 0)

        pltpu.sync_copy(rows[1], a_sh.at[bank_b.at[15]], add=True)

        @pl.when(2 * G + 3 < ngrp)
        def _():
            pltpu.async_copy(idx_hbm.at[gbase + 2 * G + 3], bank_b, sem_b)

        return carry

    lax.fori_loop(0, ngrp // 2, dgroup_step, 0)
    plsc.subcore_barrier()

    rows_per = _ASH_ROWS // _NS
    pltpu.sync_copy(
        a_sh.at[pl.ds(s * rows_per, rows_per)],
        out_hbm.at[c, pl.ds(s * rows_per, rows_per)],
    )


def kernel(x, edge_index, edge_attr, W_lin, b_lin, W_et, b_et,
           W_ih, b_ih, W_hh, b_hh, W_cls, b_cls):
    src = edge_index[0]
    dst = edge_index[1]
    gidx = edge_attr * _N + src

    pad = _EPAD - _E
    gidx_p = jnp.concatenate([gidx, jnp.zeros((pad,), jnp.int32)])
    dst_p = jnp.concatenate([dst, jnp.full((pad,), _TRASH, jnp.int32)])
    ngrp = _NCH // 8
    idx_banks = jnp.concatenate(
        [gidx_p.reshape(_NW, ngrp, 8, _CHUNK),
         dst_p.reshape(_NW, ngrp, 8, _CHUNK)], axis=2,
    ).reshape(_NW * ngrp, 16, _CHUNK)

    wlin_t = W_lin.T
    blin = b_lin.reshape(1, _D)
    wet_t = jnp.transpose(W_et, (0, 2, 1))
    bet = b_et.reshape(_T, 1, _D)
    wih_t = W_ih.T
    bih = b_ih.reshape(1, 3 * _D)
    whh_t = W_hh.T
    bhh = b_hh.reshape(1, 3 * _D)
    wcls_t = W_cls.T
    bcls = b_cls.reshape(1, 2)

    h, p = _tc_init(x, wlin_t, blin, wet_t, bet)
    for _ in range(_STEPS):
        apart = _sc_scatter(p.reshape(_T * _N, _D), idx_banks)
        h, p = _tc_gru(apart, h, wih_t, bih, whh_t, bhh, wet_t, bet)
    return _tc_readout(h, wcls_t, bcls)


# speedup vs baseline: 1.4421x; 1.3397x over previous
import functools

import jax
import jax.numpy as jnp
from jax import lax
from jax.experimental import pallas as pl
from jax.experimental.pallas import tpu as pltpu
from jax.experimental.pallas import tpu_sc as plsc

_N = 10000
_E = 320000
_D = 128
_T = 4
_STEPS = 8

_NC = 2
_NS = 16
_NW = _NC * _NS
_CHUNK = 128
_NCH = 80
_EPW = _NCH * _CHUNK
_EPAD = _NW * _EPW
_ASH_ROWS = 10240
_TRASH = _N

_NB = 2000
_GRID = _N // _NB


def _pack_rows(p):
    lo = jax.lax.bitcast_convert_type(
        p[:, : _D // 2].astype(jnp.bfloat16), jnp.uint16).astype(jnp.uint32)
    hi = jax.lax.bitcast_convert_type(
        p[:, _D // 2:].astype(jnp.bfloat16), jnp.uint16).astype(jnp.uint32)
    return jax.lax.bitcast_convert_type(lo | (hi << 16), jnp.float32)


def _init_body(x_ref, wlin_ref, blin_ref, wet_ref, bet_ref, h_ref, p_ref):
    h = jnp.dot(x_ref[...], wlin_ref[...], preferred_element_type=jnp.float32)
    h = h + blin_ref[...]
    h_ref[...] = h
    for e in range(_T):
        p_ref[e] = _pack_rows(
            jnp.dot(h, wet_ref[e], preferred_element_type=jnp.float32)
            + bet_ref[e]
        )


def _tc_init(x, wlin_t, blin, wet_t, bet):
    return pl.pallas_call(
        _init_body,
        grid=(_GRID,),
        in_specs=[
            pl.BlockSpec((_NB, _D), lambda i: (i, 0)),
            pl.BlockSpec((_D, _D), lambda i: (0, 0)),
            pl.BlockSpec((1, _D), lambda i: (0, 0)),
            pl.BlockSpec((_T, _D, _D), lambda i: (0, 0, 0)),
            pl.BlockSpec((_T, 1, _D), lambda i: (0, 0, 0)),
        ],
        out_specs=[
            pl.BlockSpec((_NB, _D), lambda i: (i, 0)),
            pl.BlockSpec((_T, _NB, _D // 2), lambda i: (0, i, 0)),
        ],
        out_shape=[
            jax.ShapeDtypeStruct((_N, _D), jnp.float32),
            jax.ShapeDtypeStruct((_T, _N, _D // 2), jnp.float32),
        ],
    )(x, wlin_t, blin, wet_t, bet)


def _gru_body(ap_ref, h_ref, wih_ref, bih_ref, whh_ref, bhh_ref,
              wet_ref, bet_ref, hout_ref, pout_ref):
    a = ap_ref[0] + ap_ref[1]
    h = h_ref[...]
    gi = jnp.dot(a, wih_ref[...], preferred_element_type=jnp.float32)
    gi = gi + bih_ref[...]
    gh = jnp.dot(h, whh_ref[...], preferred_element_type=jnp.float32)
    gh = gh + bhh_ref[...]
    r = jax.nn.sigmoid(gi[:, 0:_D] + gh[:, 0:_D])
    z = jax.nn.sigmoid(gi[:, _D:2 * _D] + gh[:, _D:2 * _D])
    n = jnp.tanh(gi[:, 2 * _D:3 * _D] + r * gh[:, 2 * _D:3 * _D])
    hn = (1.0 - z) * n + z * h
    hout_ref[...] = hn
    for e in range(_T):
        pout_ref[e] = _pack_rows(
            jnp.dot(hn, wet_ref[e], preferred_element_type=jnp.float32)
            + bet_ref[e]
        )


def _tc_gru(apart, h, wih_t, bih, whh_t, bhh, wet_t, bet):
    return pl.pallas_call(
        _gru_body,
        grid=(_GRID,),
        in_specs=[
            pl.BlockSpec((_NC, _NB, _D), lambda i: (0, i, 0)),
            pl.BlockSpec((_NB, _D), lambda i: (i, 0)),
            pl.BlockSpec((_D, 3 * _D), lambda i: (0, 0)),
            pl.BlockSpec((1, 3 * _D), lambda i: (0, 0)),
            pl.BlockSpec((_D, 3 * _D), lambda i: (0, 0)),
            pl.BlockSpec((1, 3 * _D), lambda i: (0, 0)),
            pl.BlockSpec((_T, _D, _D), lambda i: (0, 0, 0)),
            pl.BlockSpec((_T, 1, _D), lambda i: (0, 0, 0)),
        ],
        out_specs=[
            pl.BlockSpec((_NB, _D), lambda i: (i, 0)),
            pl.BlockSpec((_T, _NB, _D // 2), lambda i: (0, i, 0)),
        ],
        out_shape=[
            jax.ShapeDtypeStruct((_N, _D), jnp.float32),
            jax.ShapeDtypeStruct((_T, _N, _D // 2), jnp.float32),
        ],
    )(apart, h, wih_t, bih, whh_t, bhh, wet_t, bet)


def _readout_body(h_ref, wcls_ref, bcls_ref, out_ref):
    i = pl.program_id(0)
    s = jnp.sum(h_ref[...], axis=0, keepdims=True)
    part = jnp.dot(s, wcls_ref[...], preferred_element_type=jnp.float32)

    @pl.when(i == 0)
    def _():
        out_ref[...] = bcls_ref[...]

    out_ref[...] += part * (1.0 / _N)


def _tc_readout(h, wcls_t, bcls):
    return pl.pallas_call(
        _readout_body,
        grid=(_GRID,),
        in_specs=[
            pl.BlockSpec((_NB, _D), lambda i: (i, 0)),
            pl.BlockSpec((_D, 2), lambda i: (0, 0)),
            pl.BlockSpec((1, 2), lambda i: (0, 0)),
        ],
        out_specs=pl.BlockSpec((1, 2), lambda i: (0, 0)),
        out_shape=jax.ShapeDtypeStruct((1, 2), jnp.float32),
    )(h, wcls_t, bcls)


@functools.partial(
    pl.kernel,
    out_type=jax.ShapeDtypeStruct((_NC, _ASH_ROWS, _D), jnp.float32),
    mesh=plsc.VectorSubcoreMesh(core_axis_name="c", subcore_axis_name="s"),
    compiler_params=pltpu.CompilerParams(use_tc_tiling_on_sc=False),
    scratch_types=[
        pltpu.VMEM((16, _CHUNK), jnp.int32),
        pltpu.VMEM((16, _CHUNK), jnp.int32),
        pltpu.VMEM((_CHUNK, _D // 2), jnp.float32),
        pltpu.VMEM((_CHUNK, _D // 2), jnp.float32),
        pltpu.VMEM((_CHUNK, _D), jnp.float32),
        pltpu.VMEM((16, _D), jnp.float32),
        pltpu.VMEM_SHARED((_ASH_ROWS, _D), jnp.float32),
        pltpu.SemaphoreType.DMA,
        pltpu.SemaphoreType.DMA,
        pltpu.SemaphoreType.DMA,
        pltpu.SemaphoreType.DMA,
    ],
)
def _sc_scatter(p_hbm, idx_hbm, out_hbm,
                bank_a, bank_b, rows0, rows1, urows, zbuf, a_sh,
                gsem, sem_a, sem_b, lsem):
    rows = [rows0, rows1]
    c = lax.axis_index("c")
    s = lax.axis_index("s")
    wid = c * _NS + s
    ngrp = _NCH // 8
    gbase = wid * ngrp

    pltpu.async_copy(idx_hbm.at[gbase], bank_a, sem_a)
    pltpu.async_copy(idx_hbm.at[gbase + 1], bank_b, sem_b)
    for r in range(16):
        for q in range(8):
            zbuf[r, pl.ds(q * 16, 16)] = jnp.zeros((16,), jnp.float32)
    zrows = _ASH_ROWS // _NS
    zc = [
        pltpu.async_copy(zbuf, a_sh.at[pl.ds(s * zrows + k * 16, 16)], lsem)
        for k in range(zrows // 16)
    ]
    for copy in zc:
        copy.wait()
    plsc.subcore_barrier()


    def bank_drain(grp, bank, sem):
        pltpu.make_async_copy(idx_hbm.at[gbase + grp], bank, sem).wait()

    def gather_start(bank, q, b):
        pltpu.async_copy(p_hbm.at[bank.at[q]], rows[b], gsem)

    def gather_wait(bank, q, b):
        pltpu.make_async_copy(p_hbm.at[bank.at[q]], rows[b], gsem).wait()

    def unpack_scatter(b, ds_ref):
        src = rows[b]

        def row_fn(r, carry):
            for q in range(_D // 32):
                v = jax.lax.bitcast_convert_type(
                    src[r, pl.ds(q * 16, 16)], jnp.int32)
                lo = jax.lax.bitcast_convert_type(v << 16, jnp.float32)
                hi = jax.lax.bitcast_convert_type(
                    v & jnp.int32(-65536), jnp.float32)
                urows[r, pl.ds(q * 16, 16)] = lo
                urows[r, pl.ds(_D // 2 + q * 16, 16)] = hi
            return carry

        lax.fori_loop(0, _CHUNK, row_fn, 0)
        pltpu.sync_copy(urows, a_sh.at[ds_ref], add=True)

    bank_drain(0, bank_a, sem_a)
    gather_start(bank_a, 0, 0)

    def dgroup_step(G, carry):
        for q in range(7):
            gather_wait(bank_a, q, q % 2)
            gather_start(bank_a, q + 1, (q + 1) % 2)
            unpack_scatter(q % 2, bank_a.at[8 + q])
        gather_wait(bank_a, 7, 1)
        bank_drain(2 * G + 1, bank_b, sem_b)
        gather_start(bank_b, 0, 0)
        unpack_scatter(1, bank_a.at[15])

        @pl.when(2 * G + 2 < ngrp)
        def _():
            pltpu.async_copy(idx_hbm.at[gbase + 2 * G + 2], bank_a, sem_a)

        for q in range(7):
            gather_wait(bank_b, q, q % 2)
            gather_start(bank_b, q + 1, (q + 1) % 2)
            unpack_scatter(q % 2, bank_b.at[8 + q])
        gather_wait(bank_b, 7, 1)

        @pl.when(2 * G + 2 < ngrp)
        def _():
            bank_drain(2 * G + 2, bank_a, sem_a)
            gather_start(bank_a, 0, 0)

        unpack_scatter(1, bank_b.at[15])

        @pl.when(2 * G + 3 < ngrp)
        def _():
            pltpu.async_copy(idx_hbm.at[gbase + 2 * G + 3], bank_b, sem_b)

        return carry

    lax.fori_loop(0, ngrp // 2, dgroup_step, 0)
    plsc.subcore_barrier()

    rows_per = _ASH_ROWS // _NS
    pltpu.sync_copy(
        a_sh.at[pl.ds(s * rows_per, rows_per)],
        out_hbm.at[c, pl.ds(s * rows_per, rows_per)],
    )


def kernel(x, edge_index, edge_attr, W_lin, b_lin, W_et, b_et,
           W_ih, b_ih, W_hh, b_hh, W_cls, b_cls):
    src = edge_index[0]
    dst = edge_index[1]
    gidx = edge_attr * _N + src

    pad = _EPAD - _E
    gidx_p = jnp.concatenate([gidx, jnp.zeros((pad,), jnp.int32)])
    dst_p = jnp.concatenate([dst, jnp.full((pad,), _TRASH, jnp.int32)])
    ngrp = _NCH // 8
    idx_banks = jnp.concatenate(
        [gidx_p.reshape(_NW, ngrp, 8, _CHUNK),
         dst_p.reshape(_NW, ngrp, 8, _CHUNK)], axis=2,
    ).reshape(_NW * ngrp, 16, _CHUNK)

    wlin_t = W_lin.T
    blin = b_lin.reshape(1, _D)
    wet_t = jnp.transpose(W_et, (0, 2, 1))
    bet = b_et.reshape(_T, 1, _D)
    wih_t = W_ih.T
    bih = b_ih.reshape(1, 3 * _D)
    whh_t = W_hh.T
    bhh = b_hh.reshape(1, 3 * _D)
    wcls_t = W_cls.T
    bcls = b_cls.reshape(1, 2)

    h, p = _tc_init(x, wlin_t, blin, wet_t, bet)
    for _ in range(_STEPS):
        apart = _sc_scatter(p.reshape(_T * _N, _D // 2), idx_banks)
        h, p = _tc_gru(apart, h, wih_t, bih, whh_t, bhh, wet_t, bet)
    return _tc_readout(h, wcls_t, bcls)
